# Initial kernel scaffold; baseline (speedup 1.0000x reference)
import jax
import jax.numpy as jnp
from jax.experimental import pallas as pl


def kernel(index, output, label, target):
    raise NotImplementedError("write your pallas kernel here")


# speedup vs baseline: 1.3295x; 1.3295x over previous
import functools

import jax
import jax.numpy as jnp
from jax import lax
from jax.experimental import pallas as pl
from jax.experimental.pallas import tpu as pltpu
from jax.experimental.pallas import tpu_sc as plsc

_ALPHA = 0.3
_LAM = 3.0
_B = 4096
_C = 1000
_ROWS = 512
_CHUNK = 64

_NC = 2
_NS = 16
_NW = _NC * _NS
_B_PER_W = _B // _NW


@functools.cache
def _make_gather():
    mesh = plsc.VectorSubcoreMesh(core_axis_name="c", subcore_axis_name="s")

    @functools.partial(
        pl.kernel,
        mesh=mesh,
        out_type=jax.ShapeDtypeStruct((_B, _C), jnp.float32),
        scratch_types=[
            pltpu.VMEM((_CHUNK,), jnp.int32),
            pltpu.VMEM((_CHUNK, _C), jnp.float32),
            pltpu.SemaphoreType.DMA,
        ],
        compiler_params=pltpu.CompilerParams(use_tc_tiling_on_sc=False),
    )
    def gather_rows(table_hbm, idx_hbm, out_hbm, idx_v, rows_v, sem):
        wid = lax.axis_index("s") * _NC + lax.axis_index("c")
        base = wid * _B_PER_W
        for c in range(_B_PER_W // _CHUNK):
            off = base + c * _CHUNK
            pltpu.sync_copy(idx_hbm.at[pl.ds(off, _CHUNK)], idx_v)
            pltpu.async_copy(table_hbm.at[idx_v], rows_v, sem).wait()
            pltpu.sync_copy(rows_v, out_hbm.at[pl.ds(off, _CHUNK)])

    return gather_rows


def _loss_body(out_ref, g_ref, lab_ref, acc_ref):
    i = pl.program_id(0)
    logits = out_ref[...]
    m = jnp.max(logits, axis=1, keepdims=True)
    e = jnp.exp(logits - m)
    z = jnp.sum(e, axis=1, keepdims=True)
    y = jnp.clip(e / z, 1e-4, 1.0 - 1e-4)
    sy = jnp.sum(y, axis=1, keepdims=True)
    d2 = jnp.sum(y * y, axis=1, keepdims=True)
    d1 = jnp.sum(g_ref[...] * y, axis=1, keepdims=True)
    s = _ALPHA * d1 + (1.0 - _ALPHA) * (d2 / sy)
    elr = jnp.log(1.0 - s)
    labs = lab_ref[...]
    cls = lax.broadcasted_iota(jnp.int32, (_ROWS, _C), 1)
    lab_logit = jnp.sum(jnp.where(cls == labs, logits, 0.0), axis=1,
                        keepdims=True)
    logp = lab_logit - (m + jnp.log(z))
    part = jnp.sum(_LAM * elr - logp, keepdims=True)

    @pl.when(i == 0)
    def _():
        acc_ref[...] = jnp.zeros((1, 1), jnp.float32)

    acc_ref[...] += part

    @pl.when(i == pl.num_programs(0) - 1)
    def _():
        acc_ref[...] = acc_ref[...] * (1.0 / _B)


_loss_call = pl.pallas_call(
    _loss_body,
    grid=(_B // _ROWS,),
    in_specs=[
        pl.BlockSpec((_ROWS, _C), lambda i: (i, 0)),
        pl.BlockSpec((_ROWS, _C), lambda i: (i, 0)),
        pl.BlockSpec((_ROWS, 1), lambda i: (i, 0)),
    ],
    out_specs=pl.BlockSpec((1, 1), lambda i: (0, 0)),
    out_shape=jax.ShapeDtypeStruct((1, 1), jnp.float32),
)


def kernel(index, output, label, target):
    idx = index.astype(jnp.int32)
    gathered = _make_gather()(target, idx)
    lab2 = label.astype(jnp.int32).reshape(_B, 1)
    loss = _loss_call(output, gathered, lab2)
    return loss[0, 0]


# speedup vs baseline: 5.8470x; 4.3979x over previous
import functools

import jax
import jax.numpy as jnp
from jax import lax
from jax.experimental import pallas as pl
from jax.experimental.pallas import tpu as pltpu
from jax.experimental.pallas import tpu_sc as plsc

_ALPHA = 0.3
_LAM = 3.0
_B = 4096
_C = 1000
_ROWS = 512
_CHUNK = 64

_NC = 2
_NS = 16
_NW = _NC * _NS
_B_PER_W = _B // _NW


_W1 = 896
_W2 = 128
_W2_OFF = _C - _W2


@functools.cache
def _make_gather():
    mesh = plsc.VectorSubcoreMesh(core_axis_name="c", subcore_axis_name="s")

    @functools.partial(
        pl.kernel,
        mesh=mesh,
        out_type=(
            jax.ShapeDtypeStruct((_B, _W1), jnp.float32),
            jax.ShapeDtypeStruct((_B, _W2), jnp.float32),
        ),
        scratch_types=[
            pltpu.VMEM((_CHUNK,), jnp.int32),
            pltpu.VMEM((_CHUNK, _W1), jnp.float32),
            pltpu.VMEM((_CHUNK, _W2), jnp.float32),
            pltpu.SemaphoreType.DMA,
            pltpu.SemaphoreType.DMA,
        ],
    )
    def gather_rows(table_hbm, tail_hbm, idx_hbm, out1_hbm, out2_hbm,
                    idx_v, r1_v, r2_v, sem1, sem2):
        wid = lax.axis_index("s") * _NC + lax.axis_index("c")
        base = wid * _B_PER_W
        for c in range(_B_PER_W // _CHUNK):
            off = base + c * _CHUNK
            pltpu.sync_copy(idx_hbm.at[pl.ds(off, _CHUNK)], idx_v)
            cp1 = pltpu.async_copy(
                table_hbm.at[idx_v, pl.ds(0, _W1)], r1_v, sem1)
            cp2 = pltpu.async_copy(tail_hbm.at[idx_v], r2_v, sem2)
            cp1.wait()
            cp2.wait()
            pltpu.sync_copy(r1_v, out1_hbm.at[pl.ds(off, _CHUNK)])
            pltpu.sync_copy(r2_v, out2_hbm.at[pl.ds(off, _CHUNK)])

    return gather_rows


def _loss_body(out_ref, g1_ref, g2_ref, lab_ref, acc_ref):
    i = pl.program_id(0)
    logits = out_ref[...]
    m = jnp.max(logits, axis=1, keepdims=True)
    e = jnp.exp(logits - m)
    z = jnp.sum(e, axis=1, keepdims=True)
    y = jnp.clip(e / z, 1e-4, 1.0 - 1e-4)
    sy = jnp.sum(y, axis=1, keepdims=True)
    d2 = jnp.sum(y * y, axis=1, keepdims=True)
    tail_mask = lax.broadcasted_iota(jnp.int32, (_ROWS, _W2), 1) >= (
        _W1 - _W2_OFF)
    g2m = jnp.where(tail_mask, g2_ref[...], 0.0)
    d1 = (jnp.sum(g1_ref[...] * y[:, :_W1], axis=1, keepdims=True)
          + jnp.sum(g2m * y[:, _W2_OFF:_C], axis=1, keepdims=True))
    s = _ALPHA * d1 + (1.0 - _ALPHA) * (d2 / sy)
    elr = jnp.log(1.0 - s)
    labs = lab_ref[...]
    cls = lax.broadcasted_iota(jnp.int32, (_ROWS, _C), 1)
    lab_logit = jnp.sum(jnp.where(cls == labs, logits, 0.0), axis=1,
                        keepdims=True)
    logp = lab_logit - (m + jnp.log(z))
    part = jnp.sum(_LAM * elr - logp, keepdims=True)

    @pl.when(i == 0)
    def _():
        acc_ref[...] = jnp.zeros((1, 1), jnp.float32)

    acc_ref[...] += part

    @pl.when(i == pl.num_programs(0) - 1)
    def _():
        acc_ref[...] = acc_ref[...] * (1.0 / _B)


_loss_call = pl.pallas_call(
    _loss_body,
    grid=(_B // _ROWS,),
    in_specs=[
        pl.BlockSpec((_ROWS, _C), lambda i: (i, 0)),
        pl.BlockSpec((_ROWS, _W1), lambda i: (i, 0)),
        pl.BlockSpec((_ROWS, _W2), lambda i: (i, 0)),
        pl.BlockSpec((_ROWS, 1), lambda i: (i, 0)),
    ],
    out_specs=pl.BlockSpec((1, 1), lambda i: (0, 0)),
    out_shape=jax.ShapeDtypeStruct((1, 1), jnp.float32),
)


def kernel(index, output, label, target):
    idx = index.astype(jnp.int32)
    tail = target[:, _W2_OFF:]
    g1, g2 = _make_gather()(target, tail, idx)
    lab2 = label.astype(jnp.int32).reshape(_B, 1)
    loss = _loss_call(output, g1, g2, lab2)
    return loss[0, 0]
